# Initial kernel scaffold; baseline (speedup 1.0000x reference)
import jax
import jax.numpy as jnp
from jax.experimental import pallas as pl


def kernel(vrt, faces, vert_tri_indices, vert_tri_weights):
    raise NotImplementedError("write your pallas kernel here")


# speedup vs baseline: 28.1020x; 28.1020x over previous
import functools

import jax
import jax.numpy as jnp
from jax import lax
from jax.experimental import pallas as pl
from jax.experimental.pallas import tpu as pltpu
from jax.experimental.pallas import tpu_sc as plsc

NC = 2
NS = 16
NW = NC * NS
L = 16

CF = 128
CV = 128


def _rsqrt(x):
    i = lax.bitcast_convert_type(x, jnp.int32)
    y = lax.bitcast_convert_type(jnp.int32(0x5F3759DF) - (i >> 1), jnp.float32)
    half = jnp.float32(0.5)
    three_half = jnp.float32(1.5)
    for _ in range(2):
        y = y * (three_half - half * x * y * y)
    return y


def _normed(nx, ny, nz):
    n2 = jnp.maximum(nx * nx + ny * ny + nz * nz, jnp.float32(1e-24))
    r = _rsqrt(n2)
    return nx * r, ny * r, nz * r


def _make_stage1(F_pad, FW, W):
    mesh = plsc.VectorSubcoreMesh(core_axis_name="c", subcore_axis_name="s",
                                  num_cores=NC, num_subcores=NS)
    n_chunks = FW // CF

    @functools.partial(
        pl.kernel,
        out_type=jax.ShapeDtypeStruct((F_pad, W), jnp.float32),
        mesh=mesh,
        scratch_types=[
            pltpu.VMEM((2, 3, CF), jnp.int32),
            pltpu.VMEM((2, CF, 3), jnp.int32),
            pltpu.VMEM((2, 3, CF, W), jnp.float32),
            pltpu.VMEM((2, CF, W), jnp.float32),
            pltpu.SemaphoreType.DMA,
            pltpu.SemaphoreType.DMA,
            pltpu.SemaphoreType.DMA,
            pltpu.SemaphoreType.DMA,
        ],
        compiler_params=pltpu.CompilerParams(use_tc_tiling_on_sc=False, needs_layout_passes=False),
    )
    def k1(vrt_hbm, faces_hbm, fn_hbm, idx_v, f_v, g_v, o_v,
           sem0, sem1, osem0, osem1):
        wid = lax.axis_index("s") * NC + lax.axis_index("c")
        sems = (sem0, sem1)
        osems = (osem0, osem1)
        lanes = lax.iota(jnp.int32, 16)
        cols = [lanes * 0 + j for j in range(3)]

        def issue(slot, ch):
            base = wid * FW + ch * CF
            pltpu.sync_copy(faces_hbm.at[pl.ds(base, CF), :], f_v.at[slot])
            for g in range(CF // L):
                pos = lanes + (g * L)
                for j in range(3):
                    vj = plsc.load_gather(f_v.at[slot], [pos, cols[j]])
                    idx_v[slot, j, pl.ds(g * L, L)] = vj
            for j in range(3):
                pltpu.async_copy(vrt_hbm.at[idx_v.at[slot, j]],
                                 g_v.at[slot, j], sems[slot])

        def drain(slot):
            for j in range(3):
                pltpu.make_async_copy(vrt_hbm.at[idx_v.at[slot, j]],
                                      g_v.at[slot, j], sems[slot]).wait()

        def compute(slot, ch):
            base = wid * FW + ch * CF

            @pl.when(ch >= 2)
            def _():
                pltpu.make_async_copy(
                    o_v.at[slot], fn_hbm.at[pl.ds(wid * FW, CF)],
                    osems[slot]).wait()

            @plsc.parallel_loop(0, CF, unroll=4)
            def face(i):
                ax = g_v[slot, 0, i, pl.ds(0, L)]
                ay = g_v[slot, 0, i, pl.ds(L, L)]
                az = g_v[slot, 0, i, pl.ds(2 * L, L)]
                ux = g_v[slot, 1, i, pl.ds(0, L)] - ax
                uy = g_v[slot, 1, i, pl.ds(L, L)] - ay
                uz = g_v[slot, 1, i, pl.ds(2 * L, L)] - az
                vx = g_v[slot, 2, i, pl.ds(0, L)] - ax
                vy = g_v[slot, 2, i, pl.ds(L, L)] - ay
                vz = g_v[slot, 2, i, pl.ds(2 * L, L)] - az
                nx = uy * vz - uz * vy
                ny = uz * vx - ux * vz
                nz = ux * vy - uy * vx
                nx, ny, nz = _normed(nx, ny, nz)
                o_v[slot, i, pl.ds(0, L)] = nx
                o_v[slot, i, pl.ds(L, L)] = ny
                o_v[slot, i, pl.ds(2 * L, L)] = nz

            pltpu.async_copy(o_v.at[slot], fn_hbm.at[pl.ds(base, CF)],
                             osems[slot])

        def final_drain(slot):
            pltpu.make_async_copy(
                o_v.at[slot], fn_hbm.at[pl.ds(wid * FW, CF)],
                osems[slot]).wait()

        issue(0, 0)

        def pair(ci, carry):
            g0 = 2 * ci
            issue(1, g0 + 1)
            drain(0)
            compute(0, g0)

            @pl.when(g0 + 2 < n_chunks)
            def _():
                issue(0, g0 + 2)

            drain(1)
            compute(1, g0 + 1)
            return carry

        lax.fori_loop(0, n_chunks // 2, pair, 0)
        final_drain(0)
        final_drain(1)

    return k1


def _make_stage2(V, VW, C, W, B, F):
    mesh = plsc.VectorSubcoreMesh(core_axis_name="c", subcore_axis_name="s",
                                  num_cores=NC, num_subcores=NS)
    n_chunks = VW // CV

    @functools.partial(
        pl.kernel,
        out_type=jax.ShapeDtypeStruct((B, V, 3), jnp.float32),
        mesh=mesh,
        scratch_types=[
            pltpu.VMEM((2, C, CV), jnp.int32),
            pltpu.VMEM((2, CV, C), jnp.int32),
            pltpu.VMEM((2, CV, C), jnp.float32),
            pltpu.VMEM((2, C, CV, W), jnp.float32),
            pltpu.VMEM((2, B, CV, 3), jnp.float32),
            pltpu.SemaphoreType.DMA,
            pltpu.SemaphoreType.DMA,
            pltpu.SemaphoreType.DMA,
            pltpu.SemaphoreType.DMA,
        ],
        compiler_params=pltpu.CompilerParams(use_tc_tiling_on_sc=False, needs_layout_passes=False),
    )
    def k2(fn_hbm, vti_hbm, vtw_hbm, out_hbm, idx_v, ti_v, tw_v, g_v, o_v,
           sem0, sem1, osem0, osem1):
        wid = lax.axis_index("s") * NC + lax.axis_index("c")
        sems = (sem0, sem1)
        osems = (osem0, osem1)
        lanes = lax.iota(jnp.int32, 16)
        c0 = lanes * 0
        c1 = c0 + 1
        c2 = c0 + 2
        cols = [c0 + j for j in range(C)]
        sentinel = c0 + F
        wzero = sentinel.astype(jnp.float32) * 0.0

        def issue(slot, ch):
            base = wid * VW + ch * CV
            pltpu.sync_copy(vti_hbm.at[pl.ds(base, CV), :], ti_v.at[slot])
            pltpu.sync_copy(vtw_hbm.at[pl.ds(base, CV), :], tw_v.at[slot])
            for g in range(CV // L):
                pos = lanes + (g * L)
                for j in range(C):
                    vj = plsc.load_gather(ti_v.at[slot], [pos, cols[j]])
                    wj = plsc.load_gather(tw_v.at[slot], [pos, cols[j]])
                    vj = jnp.where(wj == wzero, sentinel, vj)
                    idx_v[slot, j, pl.ds(g * L, L)] = vj
            for j in range(C):
                pltpu.async_copy(fn_hbm.at[idx_v.at[slot, j]],
                                 g_v.at[slot, j], sems[slot])

        def drain(slot):
            for j in range(C):
                pltpu.make_async_copy(fn_hbm.at[idx_v.at[slot, j]],
                                      g_v.at[slot, j], sems[slot]).wait()

        def compute(slot, ch):
            base = wid * VW + ch * CV

            @pl.when(ch >= 2)
            def _():
                pltpu.make_async_copy(
                    o_v.at[slot],
                    out_hbm.at[:, pl.ds(wid * VW, CV), :],
                    osems[slot]).wait()

            @plsc.parallel_loop(0, CV, unroll=4)
            def vert(i):
                sx = g_v[slot, 0, i, pl.ds(0, L)]
                sy = g_v[slot, 0, i, pl.ds(L, L)]
                sz = g_v[slot, 0, i, pl.ds(2 * L, L)]
                for j in range(1, C):
                    sx = sx + g_v[slot, j, i, pl.ds(0, L)]
                    sy = sy + g_v[slot, j, i, pl.ds(L, L)]
                    sz = sz + g_v[slot, j, i, pl.ds(2 * L, L)]
                sx, sy, sz = _normed(sx, sy, sz)
                row = jnp.broadcast_to(i, (16,)).astype(jnp.int32)
                plsc.store_scatter(o_v.at[slot], [lanes, row, c0], sx)
                plsc.store_scatter(o_v.at[slot], [lanes, row, c1], sy)
                plsc.store_scatter(o_v.at[slot], [lanes, row, c2], sz)

            pltpu.async_copy(o_v.at[slot],
                             out_hbm.at[:, pl.ds(base, CV), :], osems[slot])

        def final_drain(slot):
            pltpu.make_async_copy(
                o_v.at[slot], out_hbm.at[:, pl.ds(wid * VW, CV), :],
                osems[slot]).wait()

        issue(0, 0)

        def pair(ci, carry):
            g0 = 2 * ci
            issue(1, g0 + 1)
            drain(0)
            compute(0, g0)

            @pl.when(g0 + 2 < n_chunks)
            def _():
                issue(0, g0 + 2)

            drain(1)
            compute(1, g0 + 1)
            return carry

        lax.fori_loop(0, n_chunks // 2, pair, 0)
        final_drain(0)
        final_drain(1)

    return k2


def kernel(vrt, faces, vert_tri_indices, vert_tri_weights):
    B, V, _ = vrt.shape
    F = faces.shape[0]
    C = vert_tri_indices.shape[1]
    W = 3 * B

    F_pad = -(-(F + 1) // (NW * CF)) * (NW * CF)
    FW = F_pad // NW
    VW = V // NW

    vrt_t = jnp.transpose(vrt, (1, 2, 0)).reshape(V, W)
    faces_p = (jnp.zeros((F_pad, 3), jnp.int32)
               .at[:F].set(faces.astype(jnp.int32)))
    vti_r = vert_tri_indices.astype(jnp.int32)
    vtw_r = vert_tri_weights.reshape(V, C)

    fn = _make_stage1(F_pad, FW, W)(vrt_t, faces_p)
    return _make_stage2(V, VW, C, W, B, F)(fn, vti_r, vtw_r)


# speedup vs baseline: 87.8727x; 3.1269x over previous
import functools

import jax
import jax.numpy as jnp
from jax import lax
from jax.experimental import pallas as pl
from jax.experimental.pallas import tpu as pltpu
from jax.experimental.pallas import tpu_sc as plsc

NC = 2
NS = 16
NW = NC * NS
L = 16

CF = 128
CV = 128


def _rsqrt(x):
    i = lax.bitcast_convert_type(x, jnp.int32)
    y = lax.bitcast_convert_type(jnp.int32(0x5F3759DF) - (i >> 1), jnp.float32)
    half = jnp.float32(0.5)
    three_half = jnp.float32(1.5)
    for _ in range(2):
        y = y * (three_half - half * x * y * y)
    return y


def _normed(nx, ny, nz):
    n2 = jnp.maximum(nx * nx + ny * ny + nz * nz, jnp.float32(1e-24))
    r = _rsqrt(n2)
    return nx * r, ny * r, nz * r


def _make_stage1(F_pad, FW, W):
    mesh = plsc.VectorSubcoreMesh(core_axis_name="c", subcore_axis_name="s",
                                  num_cores=NC, num_subcores=NS)
    n_chunks = FW // CF

    @functools.partial(
        pl.kernel,
        out_type=jax.ShapeDtypeStruct((F_pad, W), jnp.float32),
        mesh=mesh,
        scratch_types=[
            pltpu.VMEM((2, 3, CF), jnp.int32),
            pltpu.VMEM((2, 3, CF, W), jnp.float32),
            pltpu.VMEM((2, CF, W), jnp.float32),
            pltpu.SemaphoreType.DMA,
            pltpu.SemaphoreType.DMA,
            pltpu.SemaphoreType.DMA,
            pltpu.SemaphoreType.DMA,
            pltpu.SemaphoreType.DMA,
            pltpu.SemaphoreType.DMA,
        ],
        compiler_params=pltpu.CompilerParams(use_tc_tiling_on_sc=False),
    )
    def k1(vrt_hbm, faces_hbm, fn_hbm, idx_v, g_v, o_v,
           sem0, sem1, osem0, osem1, isem0, isem1):
        wid = lax.axis_index("s") * NC + lax.axis_index("c")
        sems = (sem0, sem1)
        osems = (osem0, osem1)
        isems = (isem0, isem1)

        def stage_raw(slot, ch):
            base = wid * FW + ch * CF
            pltpu.async_copy(faces_hbm.at[:, pl.ds(base, CF)],
                             idx_v.at[slot], isems[slot])

        def fire(slot, ch):
            base = wid * FW + ch * CF
            pltpu.make_async_copy(faces_hbm.at[:, pl.ds(base, CF)],
                                  idx_v.at[slot], isems[slot]).wait()
            for j in range(3):
                pltpu.async_copy(vrt_hbm.at[idx_v.at[slot, j]],
                                 g_v.at[slot, j], sems[slot])

        def drain(slot):
            for j in range(3):
                pltpu.make_async_copy(vrt_hbm.at[idx_v.at[slot, j]],
                                      g_v.at[slot, j], sems[slot]).wait()

        def compute(slot, ch):
            base = wid * FW + ch * CF

            @pl.when(ch >= 2)
            def _():
                pltpu.make_async_copy(
                    o_v.at[slot], fn_hbm.at[pl.ds(wid * FW, CF)],
                    osems[slot]).wait()

            @plsc.parallel_loop(0, CF, unroll=4)
            def face(i):
                ax = g_v[slot, 0, i, pl.ds(0, L)]
                ay = g_v[slot, 0, i, pl.ds(L, L)]
                az = g_v[slot, 0, i, pl.ds(2 * L, L)]
                ux = g_v[slot, 1, i, pl.ds(0, L)] - ax
                uy = g_v[slot, 1, i, pl.ds(L, L)] - ay
                uz = g_v[slot, 1, i, pl.ds(2 * L, L)] - az
                vx = g_v[slot, 2, i, pl.ds(0, L)] - ax
                vy = g_v[slot, 2, i, pl.ds(L, L)] - ay
                vz = g_v[slot, 2, i, pl.ds(2 * L, L)] - az
                nx = uy * vz - uz * vy
                ny = uz * vx - ux * vz
                nz = ux * vy - uy * vx
                nx, ny, nz = _normed(nx, ny, nz)
                o_v[slot, i, pl.ds(0, L)] = nx
                o_v[slot, i, pl.ds(L, L)] = ny
                o_v[slot, i, pl.ds(2 * L, L)] = nz

            pltpu.async_copy(o_v.at[slot], fn_hbm.at[pl.ds(base, CF)],
                             osems[slot])

        def final_drain(slot):
            pltpu.make_async_copy(
                o_v.at[slot], fn_hbm.at[pl.ds(wid * FW, CF)],
                osems[slot]).wait()

        stage_raw(0, 0)
        fire(0, 0)
        stage_raw(1, 1)

        def pair(ci, carry):
            g0 = 2 * ci
            fire(1, g0 + 1)
            drain(0)

            @pl.when(g0 + 2 < n_chunks)
            def _():
                stage_raw(0, g0 + 2)

            compute(0, g0)

            @pl.when(g0 + 2 < n_chunks)
            def _():
                fire(0, g0 + 2)

            drain(1)

            @pl.when(g0 + 3 < n_chunks)
            def _():
                stage_raw(1, g0 + 3)

            compute(1, g0 + 1)
            return carry

        lax.fori_loop(0, n_chunks // 2, pair, 0)
        final_drain(0)
        final_drain(1)

    return k1


def _make_stage2(V, VW, C, W):
    mesh = plsc.VectorSubcoreMesh(core_axis_name="c", subcore_axis_name="s",
                                  num_cores=NC, num_subcores=NS)
    n_chunks = VW // CV

    @functools.partial(
        pl.kernel,
        out_type=jax.ShapeDtypeStruct((V, W), jnp.float32),
        mesh=mesh,
        scratch_types=[
            pltpu.VMEM((2, C, CV), jnp.int32),
            pltpu.VMEM((2, C, CV, W), jnp.float32),
            pltpu.VMEM((2, CV, W), jnp.float32),
            pltpu.SemaphoreType.DMA,
            pltpu.SemaphoreType.DMA,
            pltpu.SemaphoreType.DMA,
            pltpu.SemaphoreType.DMA,
            pltpu.SemaphoreType.DMA,
            pltpu.SemaphoreType.DMA,
        ],
        compiler_params=pltpu.CompilerParams(use_tc_tiling_on_sc=False),
    )
    def k2(fn_hbm, vti_hbm, out_hbm, idx_v, g_v, o_v,
           sem0, sem1, osem0, osem1, isem0, isem1):
        wid = lax.axis_index("s") * NC + lax.axis_index("c")
        sems = (sem0, sem1)
        osems = (osem0, osem1)
        isems = (isem0, isem1)

        def stage_raw(slot, ch):
            base = wid * VW + ch * CV
            pltpu.async_copy(vti_hbm.at[:, pl.ds(base, CV)],
                             idx_v.at[slot], isems[slot])

        def fire(slot, ch):
            base = wid * VW + ch * CV
            pltpu.make_async_copy(vti_hbm.at[:, pl.ds(base, CV)],
                                  idx_v.at[slot], isems[slot]).wait()
            for j in range(C):
                pltpu.async_copy(fn_hbm.at[idx_v.at[slot, j]],
                                 g_v.at[slot, j], sems[slot])

        def drain(slot):
            for j in range(C):
                pltpu.make_async_copy(fn_hbm.at[idx_v.at[slot, j]],
                                      g_v.at[slot, j], sems[slot]).wait()

        def compute(slot, ch):
            base = wid * VW + ch * CV

            @pl.when(ch >= 2)
            def _():
                pltpu.make_async_copy(
                    o_v.at[slot], out_hbm.at[pl.ds(wid * VW, CV)],
                    osems[slot]).wait()

            @plsc.parallel_loop(0, CV, unroll=4)
            def vert(i):
                sx = g_v[slot, 0, i, pl.ds(0, L)]
                sy = g_v[slot, 0, i, pl.ds(L, L)]
                sz = g_v[slot, 0, i, pl.ds(2 * L, L)]
                for j in range(1, C):
                    sx = sx + g_v[slot, j, i, pl.ds(0, L)]
                    sy = sy + g_v[slot, j, i, pl.ds(L, L)]
                    sz = sz + g_v[slot, j, i, pl.ds(2 * L, L)]
                sx, sy, sz = _normed(sx, sy, sz)
                o_v[slot, i, pl.ds(0, L)] = sx
                o_v[slot, i, pl.ds(L, L)] = sy
                o_v[slot, i, pl.ds(2 * L, L)] = sz

            pltpu.async_copy(o_v.at[slot], out_hbm.at[pl.ds(base, CV)],
                             osems[slot])

        def final_drain(slot):
            pltpu.make_async_copy(
                o_v.at[slot], out_hbm.at[pl.ds(wid * VW, CV)],
                osems[slot]).wait()

        stage_raw(0, 0)
        fire(0, 0)
        stage_raw(1, 1)

        def pair(ci, carry):
            g0 = 2 * ci
            fire(1, g0 + 1)
            drain(0)

            @pl.when(g0 + 2 < n_chunks)
            def _():
                stage_raw(0, g0 + 2)

            compute(0, g0)

            @pl.when(g0 + 2 < n_chunks)
            def _():
                fire(0, g0 + 2)

            drain(1)

            @pl.when(g0 + 3 < n_chunks)
            def _():
                stage_raw(1, g0 + 3)

            compute(1, g0 + 1)
            return carry

        lax.fori_loop(0, n_chunks // 2, pair, 0)
        final_drain(0)
        final_drain(1)

    return k2


def kernel(vrt, faces, vert_tri_indices, vert_tri_weights):
    B, V, _ = vrt.shape
    F = faces.shape[0]
    C = vert_tri_indices.shape[1]
    W = 3 * B

    F_pad = -(-(F + 1) // (NW * CF)) * (NW * CF)
    FW = F_pad // NW
    VW = V // NW

    vrt_t = jnp.transpose(vrt, (1, 2, 0)).reshape(V, W)
    faces_t = (jnp.zeros((3, F_pad), jnp.int32)
               .at[:, :F].set(faces.T.astype(jnp.int32)))
    w = vert_tri_weights.reshape(V, C)
    vti_t = jnp.where(w != 0, vert_tri_indices.astype(jnp.int32),
                      jnp.int32(F)).T

    fn = _make_stage1(F_pad, FW, W)(vrt_t, faces_t)
    out_t = _make_stage2(V, VW, C, W)(fn, vti_t)
    return out_t.reshape(V, 3, B).transpose(2, 0, 1)


# speedup vs baseline: 87.9271x; 1.0006x over previous
import functools

import jax
import jax.numpy as jnp
from jax import lax
from jax.experimental import pallas as pl
from jax.experimental.pallas import tpu as pltpu
from jax.experimental.pallas import tpu_sc as plsc

NC = 2
NS = 16
NW = NC * NS
L = 16

CF = 128
CV = 128


def _rsqrt(x):
    i = lax.bitcast_convert_type(x, jnp.int32)
    y = lax.bitcast_convert_type(jnp.int32(0x5F3759DF) - (i >> 1), jnp.float32)
    half = jnp.float32(0.5)
    three_half = jnp.float32(1.5)
    for _ in range(2):
        y = y * (three_half - half * x * y * y)
    return y


def _normed(nx, ny, nz):
    n2 = jnp.maximum(nx * nx + ny * ny + nz * nz, jnp.float32(1e-24))
    r = _rsqrt(n2)
    return nx * r, ny * r, nz * r


def _make_stage1(F_pad, FW, W):
    mesh = plsc.VectorSubcoreMesh(core_axis_name="c", subcore_axis_name="s",
                                  num_cores=NC, num_subcores=NS)
    n_chunks = FW // CF

    @functools.partial(
        pl.kernel,
        out_type=jax.ShapeDtypeStruct((F_pad, W), jnp.float32),
        mesh=mesh,
        scratch_types=[
            pltpu.VMEM((2, 3, CF), jnp.int32),
            pltpu.VMEM((2, 3, CF, W), jnp.float32),
            pltpu.VMEM((2, CF, W), jnp.float32),
            pltpu.SemaphoreType.DMA,
            pltpu.SemaphoreType.DMA,
            pltpu.SemaphoreType.DMA,
            pltpu.SemaphoreType.DMA,
            pltpu.SemaphoreType.DMA,
            pltpu.SemaphoreType.DMA,
        ],
        compiler_params=pltpu.CompilerParams(use_tc_tiling_on_sc=False),
    )
    def k1(vrt_hbm, faces_hbm, fn_hbm, idx_v, g_v, o_v,
           sem0, sem1, osem0, osem1, isem0, isem1):
        wid = lax.axis_index("s") * NC + lax.axis_index("c")
        sems = (sem0, sem1)
        osems = (osem0, osem1)
        isems = (isem0, isem1)

        def stage_raw(slot, ch):
            base = wid * FW + ch * CF
            pltpu.async_copy(faces_hbm.at[:, pl.ds(base, CF)],
                             idx_v.at[slot], isems[slot])

        def fire(slot, ch):
            base = wid * FW + ch * CF
            pltpu.make_async_copy(faces_hbm.at[:, pl.ds(base, CF)],
                                  idx_v.at[slot], isems[slot]).wait()
            for j in range(3):
                for h in range(2):
                    pltpu.async_copy(
                        vrt_hbm.at[idx_v.at[slot, j, pl.ds(h * (CF // 2),
                                                           CF // 2)]],
                        g_v.at[slot, j, pl.ds(h * (CF // 2), CF // 2)],
                        sems[slot])

        def drain(slot):
            for j in range(3):
                pltpu.make_async_copy(vrt_hbm.at[idx_v.at[slot, j]],
                                      g_v.at[slot, j], sems[slot]).wait()

        def compute(slot, ch):
            base = wid * FW + ch * CF

            @pl.when(ch >= 2)
            def _():
                pltpu.make_async_copy(
                    o_v.at[slot], fn_hbm.at[pl.ds(wid * FW, CF)],
                    osems[slot]).wait()

            @plsc.parallel_loop(0, CF, unroll=4)
            def face(i):
                ax = g_v[slot, 0, i, pl.ds(0, L)]
                ay = g_v[slot, 0, i, pl.ds(L, L)]
                az = g_v[slot, 0, i, pl.ds(2 * L, L)]
                ux = g_v[slot, 1, i, pl.ds(0, L)] - ax
                uy = g_v[slot, 1, i, pl.ds(L, L)] - ay
                uz = g_v[slot, 1, i, pl.ds(2 * L, L)] - az
                vx = g_v[slot, 2, i, pl.ds(0, L)] - ax
                vy = g_v[slot, 2, i, pl.ds(L, L)] - ay
                vz = g_v[slot, 2, i, pl.ds(2 * L, L)] - az
                nx = uy * vz - uz * vy
                ny = uz * vx - ux * vz
                nz = ux * vy - uy * vx
                nx, ny, nz = _normed(nx, ny, nz)
                o_v[slot, i, pl.ds(0, L)] = nx
                o_v[slot, i, pl.ds(L, L)] = ny
                o_v[slot, i, pl.ds(2 * L, L)] = nz

            pltpu.async_copy(o_v.at[slot], fn_hbm.at[pl.ds(base, CF)],
                             osems[slot])

        def final_drain(slot):
            pltpu.make_async_copy(
                o_v.at[slot], fn_hbm.at[pl.ds(wid * FW, CF)],
                osems[slot]).wait()

        stage_raw(0, 0)
        fire(0, 0)
        stage_raw(1, 1)

        def pair(ci, carry):
            g0 = 2 * ci
            fire(1, g0 + 1)
            drain(0)

            @pl.when(g0 + 2 < n_chunks)
            def _():
                stage_raw(0, g0 + 2)

            compute(0, g0)

            @pl.when(g0 + 2 < n_chunks)
            def _():
                fire(0, g0 + 2)

            drain(1)

            @pl.when(g0 + 3 < n_chunks)
            def _():
                stage_raw(1, g0 + 3)

            compute(1, g0 + 1)
            return carry

        lax.fori_loop(0, n_chunks // 2, pair, 0)
        final_drain(0)
        final_drain(1)

    return k1


def _make_stage2(V, VW, C, W):
    mesh = plsc.VectorSubcoreMesh(core_axis_name="c", subcore_axis_name="s",
                                  num_cores=NC, num_subcores=NS)
    n_chunks = VW // CV

    @functools.partial(
        pl.kernel,
        out_type=jax.ShapeDtypeStruct((V, W), jnp.float32),
        mesh=mesh,
        scratch_types=[
            pltpu.VMEM((2, C, CV), jnp.int32),
            pltpu.VMEM((2, C, CV, W), jnp.float32),
            pltpu.VMEM((2, CV, W), jnp.float32),
            pltpu.SemaphoreType.DMA,
            pltpu.SemaphoreType.DMA,
            pltpu.SemaphoreType.DMA,
            pltpu.SemaphoreType.DMA,
            pltpu.SemaphoreType.DMA,
            pltpu.SemaphoreType.DMA,
        ],
        compiler_params=pltpu.CompilerParams(use_tc_tiling_on_sc=False),
    )
    def k2(fn_hbm, vti_hbm, out_hbm, idx_v, g_v, o_v,
           sem0, sem1, osem0, osem1, isem0, isem1):
        wid = lax.axis_index("s") * NC + lax.axis_index("c")
        sems = (sem0, sem1)
        osems = (osem0, osem1)
        isems = (isem0, isem1)

        def stage_raw(slot, ch):
            base = wid * VW + ch * CV
            pltpu.async_copy(vti_hbm.at[:, pl.ds(base, CV)],
                             idx_v.at[slot], isems[slot])

        def fire(slot, ch):
            base = wid * VW + ch * CV
            pltpu.make_async_copy(vti_hbm.at[:, pl.ds(base, CV)],
                                  idx_v.at[slot], isems[slot]).wait()
            for j in range(C):
                pltpu.async_copy(fn_hbm.at[idx_v.at[slot, j]],
                                 g_v.at[slot, j], sems[slot])

        def drain(slot):
            for j in range(C):
                pltpu.make_async_copy(fn_hbm.at[idx_v.at[slot, j]],
                                      g_v.at[slot, j], sems[slot]).wait()

        def compute(slot, ch):
            base = wid * VW + ch * CV

            @pl.when(ch >= 2)
            def _():
                pltpu.make_async_copy(
                    o_v.at[slot], out_hbm.at[pl.ds(wid * VW, CV)],
                    osems[slot]).wait()

            @plsc.parallel_loop(0, CV, unroll=4)
            def vert(i):
                sx = g_v[slot, 0, i, pl.ds(0, L)]
                sy = g_v[slot, 0, i, pl.ds(L, L)]
                sz = g_v[slot, 0, i, pl.ds(2 * L, L)]
                for j in range(1, C):
                    sx = sx + g_v[slot, j, i, pl.ds(0, L)]
                    sy = sy + g_v[slot, j, i, pl.ds(L, L)]
                    sz = sz + g_v[slot, j, i, pl.ds(2 * L, L)]
                sx, sy, sz = _normed(sx, sy, sz)
                o_v[slot, i, pl.ds(0, L)] = sx
                o_v[slot, i, pl.ds(L, L)] = sy
                o_v[slot, i, pl.ds(2 * L, L)] = sz

            pltpu.async_copy(o_v.at[slot], out_hbm.at[pl.ds(base, CV)],
                             osems[slot])

        def final_drain(slot):
            pltpu.make_async_copy(
                o_v.at[slot], out_hbm.at[pl.ds(wid * VW, CV)],
                osems[slot]).wait()

        stage_raw(0, 0)
        fire(0, 0)
        stage_raw(1, 1)

        def pair(ci, carry):
            g0 = 2 * ci
            fire(1, g0 + 1)
            drain(0)

            @pl.when(g0 + 2 < n_chunks)
            def _():
                stage_raw(0, g0 + 2)

            compute(0, g0)

            @pl.when(g0 + 2 < n_chunks)
            def _():
                fire(0, g0 + 2)

            drain(1)

            @pl.when(g0 + 3 < n_chunks)
            def _():
                stage_raw(1, g0 + 3)

            compute(1, g0 + 1)
            return carry

        lax.fori_loop(0, n_chunks // 2, pair, 0)
        final_drain(0)
        final_drain(1)

    return k2


def kernel(vrt, faces, vert_tri_indices, vert_tri_weights):
    B, V, _ = vrt.shape
    F = faces.shape[0]
    C = vert_tri_indices.shape[1]
    W = 3 * B

    F_pad = -(-(F + 1) // (NW * CF)) * (NW * CF)
    FW = F_pad // NW
    VW = V // NW

    vrt_t = jnp.transpose(vrt, (1, 2, 0)).reshape(V, W)
    faces_t = (jnp.zeros((3, F_pad), jnp.int32)
               .at[:, :F].set(faces.T.astype(jnp.int32)))
    w = vert_tri_weights.reshape(V, C)
    vti_t = jnp.where(w != 0, vert_tri_indices.astype(jnp.int32),
                      jnp.int32(F)).T

    fn = _make_stage1(F_pad, FW, W)(vrt_t, faces_t)
    out_t = _make_stage2(V, VW, C, W)(fn, vti_t)
    return out_t.reshape(V, 3, B).transpose(2, 0, 1)


# speedup vs baseline: 92.7834x; 1.0552x over previous
import functools

import jax
import jax.numpy as jnp
from jax import lax
from jax.experimental import pallas as pl
from jax.experimental.pallas import tpu as pltpu
from jax.experimental.pallas import tpu_sc as plsc

NC = 2
NS = 16
NW = NC * NS
L = 16
NB = 4

CF = 128
CV = 64


def _rsqrt(x):
    i = lax.bitcast_convert_type(x, jnp.int32)
    y = lax.bitcast_convert_type(jnp.int32(0x5F3759DF) - (i >> 1), jnp.float32)
    half = jnp.float32(0.5)
    three_half = jnp.float32(1.5)
    for _ in range(2):
        y = y * (three_half - half * x * y * y)
    return y


def _normed(nx, ny, nz):
    n2 = jnp.maximum(nx * nx + ny * ny + nz * nz, jnp.float32(1e-24))
    r = _rsqrt(n2)
    return nx * r, ny * r, nz * r


def _ring(n_chunks, stage_raw, fire, drain, compute, final_drain):
    stage_raw(0, 0)
    stage_raw(1, 1)
    fire(0, 0)
    stage_raw(2, 2)
    fire(1, 1)
    stage_raw(3, 3)

    def quad(ci, carry):
        g0 = 4 * ci
        for b in range(NB):
            g = g0 + b
            drain(b)

            @pl.when(g + NB < n_chunks)
            def _(b=b, g=g):
                stage_raw(b, g + NB)

            @pl.when(g + 2 < n_chunks)
            def _(b=b, g=g):
                fire((b + 2) % NB, g + 2)

            compute(b, g)
        return carry

    lax.fori_loop(0, n_chunks // NB, quad, 0)
    for b in range(NB):
        final_drain(b)


def _make_stage1(F_pad, FW, W):
    mesh = plsc.VectorSubcoreMesh(core_axis_name="c", subcore_axis_name="s",
                                  num_cores=NC, num_subcores=NS)
    n_chunks = FW // CF

    @functools.partial(
        pl.kernel,
        out_type=jax.ShapeDtypeStruct((F_pad, W), jnp.float32),
        mesh=mesh,
        scratch_types=[
            pltpu.VMEM((NB, 3, CF), jnp.int32),
            pltpu.VMEM((NB, 3, CF, W), jnp.float32),
            pltpu.VMEM((NB, CF, W), jnp.float32),
        ] + [pltpu.SemaphoreType.DMA] * (3 * NB),
        compiler_params=pltpu.CompilerParams(use_tc_tiling_on_sc=False),
    )
    def k1(vrt_hbm, faces_hbm, fn_hbm, idx_v, g_v, o_v, *sems_all):
        wid = lax.axis_index("s") * NC + lax.axis_index("c")
        sems = sems_all[0:NB]
        osems = sems_all[NB:2 * NB]
        isems = sems_all[2 * NB:3 * NB]

        def stage_raw(slot, ch):
            base = wid * FW + ch * CF
            pltpu.async_copy(faces_hbm.at[:, pl.ds(base, CF)],
                             idx_v.at[slot], isems[slot])

        def fire(slot, ch):
            base = wid * FW + ch * CF
            pltpu.make_async_copy(faces_hbm.at[:, pl.ds(base, CF)],
                                  idx_v.at[slot], isems[slot]).wait()
            for j in range(3):
                pltpu.async_copy(vrt_hbm.at[idx_v.at[slot, j]],
                                 g_v.at[slot, j], sems[slot])

        def drain(slot):
            for j in range(3):
                pltpu.make_async_copy(vrt_hbm.at[idx_v.at[slot, j]],
                                      g_v.at[slot, j], sems[slot]).wait()

        def compute(slot, ch):
            base = wid * FW + ch * CF

            @pl.when(ch >= NB)
            def _():
                pltpu.make_async_copy(
                    o_v.at[slot], fn_hbm.at[pl.ds(wid * FW, CF)],
                    osems[slot]).wait()

            @plsc.parallel_loop(0, CF, unroll=4)
            def face(i):
                ax = g_v[slot, 0, i, pl.ds(0, L)]
                ay = g_v[slot, 0, i, pl.ds(L, L)]
                az = g_v[slot, 0, i, pl.ds(2 * L, L)]
                ux = g_v[slot, 1, i, pl.ds(0, L)] - ax
                uy = g_v[slot, 1, i, pl.ds(L, L)] - ay
                uz = g_v[slot, 1, i, pl.ds(2 * L, L)] - az
                vx = g_v[slot, 2, i, pl.ds(0, L)] - ax
                vy = g_v[slot, 2, i, pl.ds(L, L)] - ay
                vz = g_v[slot, 2, i, pl.ds(2 * L, L)] - az
                nx = uy * vz - uz * vy
                ny = uz * vx - ux * vz
                nz = ux * vy - uy * vx
                nx, ny, nz = _normed(nx, ny, nz)
                o_v[slot, i, pl.ds(0, L)] = nx
                o_v[slot, i, pl.ds(L, L)] = ny
                o_v[slot, i, pl.ds(2 * L, L)] = nz

            pltpu.async_copy(o_v.at[slot], fn_hbm.at[pl.ds(base, CF)],
                             osems[slot])

        def final_drain(slot):
            pltpu.make_async_copy(
                o_v.at[slot], fn_hbm.at[pl.ds(wid * FW, CF)],
                osems[slot]).wait()

        _ring(n_chunks, stage_raw, fire, drain, compute, final_drain)

    return k1


def _make_stage2(V, VW, C, W):
    mesh = plsc.VectorSubcoreMesh(core_axis_name="c", subcore_axis_name="s",
                                  num_cores=NC, num_subcores=NS)
    n_chunks = VW // CV

    @functools.partial(
        pl.kernel,
        out_type=jax.ShapeDtypeStruct((V, W), jnp.float32),
        mesh=mesh,
        scratch_types=[
            pltpu.VMEM((NB, C, CV), jnp.int32),
            pltpu.VMEM((NB, C, CV, W), jnp.float32),
            pltpu.VMEM((NB, CV, W), jnp.float32),
        ] + [pltpu.SemaphoreType.DMA] * (3 * NB),
        compiler_params=pltpu.CompilerParams(use_tc_tiling_on_sc=False),
    )
    def k2(fn_hbm, vti_hbm, out_hbm, idx_v, g_v, o_v, *sems_all):
        wid = lax.axis_index("s") * NC + lax.axis_index("c")
        sems = sems_all[0:NB]
        osems = sems_all[NB:2 * NB]
        isems = sems_all[2 * NB:3 * NB]

        def stage_raw(slot, ch):
            base = wid * VW + ch * CV
            pltpu.async_copy(vti_hbm.at[:, pl.ds(base, CV)],
                             idx_v.at[slot], isems[slot])

        def fire(slot, ch):
            base = wid * VW + ch * CV
            pltpu.make_async_copy(vti_hbm.at[:, pl.ds(base, CV)],
                                  idx_v.at[slot], isems[slot]).wait()
            for j in range(C):
                pltpu.async_copy(fn_hbm.at[idx_v.at[slot, j]],
                                 g_v.at[slot, j], sems[slot])

        def drain(slot):
            for j in range(C):
                pltpu.make_async_copy(fn_hbm.at[idx_v.at[slot, j]],
                                      g_v.at[slot, j], sems[slot]).wait()

        def compute(slot, ch):
            base = wid * VW + ch * CV

            @pl.when(ch >= NB)
            def _():
                pltpu.make_async_copy(
                    o_v.at[slot], out_hbm.at[pl.ds(wid * VW, CV)],
                    osems[slot]).wait()

            @plsc.parallel_loop(0, CV, unroll=4)
            def vert(i):
                sx = g_v[slot, 0, i, pl.ds(0, L)]
                sy = g_v[slot, 0, i, pl.ds(L, L)]
                sz = g_v[slot, 0, i, pl.ds(2 * L, L)]
                for j in range(1, C):
                    sx = sx + g_v[slot, j, i, pl.ds(0, L)]
                    sy = sy + g_v[slot, j, i, pl.ds(L, L)]
                    sz = sz + g_v[slot, j, i, pl.ds(2 * L, L)]
                sx, sy, sz = _normed(sx, sy, sz)
                o_v[slot, i, pl.ds(0, L)] = sx
                o_v[slot, i, pl.ds(L, L)] = sy
                o_v[slot, i, pl.ds(2 * L, L)] = sz

            pltpu.async_copy(o_v.at[slot], out_hbm.at[pl.ds(base, CV)],
                             osems[slot])

        def final_drain(slot):
            pltpu.make_async_copy(
                o_v.at[slot], out_hbm.at[pl.ds(wid * VW, CV)],
                osems[slot]).wait()

        _ring(n_chunks, stage_raw, fire, drain, compute, final_drain)

    return k2


def kernel(vrt, faces, vert_tri_indices, vert_tri_weights):
    B, V, _ = vrt.shape
    F = faces.shape[0]
    C = vert_tri_indices.shape[1]
    W = 3 * B

    F_pad = -(-(F + 1) // (NW * CF)) * (NW * CF)
    FW = F_pad // NW
    VW = V // NW

    vrt_t = jnp.transpose(vrt, (1, 2, 0)).reshape(V, W)
    faces_t = (jnp.zeros((3, F_pad), jnp.int32)
               .at[:, :F].set(faces.T.astype(jnp.int32)))
    w = vert_tri_weights.reshape(V, C)
    vti_t = jnp.where(w != 0, vert_tri_indices.astype(jnp.int32),
                      jnp.int32(F)).T

    fn = _make_stage1(F_pad, FW, W)(vrt_t, faces_t)
    out_t = _make_stage2(V, VW, C, W)(fn, vti_t)
    return out_t.reshape(V, 3, B).transpose(2, 0, 1)
